# baseline (speedup 1.0000x reference)
import functools

import jax
import jax.numpy as jnp
from jax import lax
from jax.experimental import pallas as pl
from jax.experimental.pallas import tpu as pltpu
from jax.experimental.pallas import tpu_sc as plsc

_D = 32
_CH = 128


@functools.cache
def _make_lookup(B: int, V: int):
    info = plsc.get_sparse_core_info()
    nc, ns = info.num_cores, info.num_subcores
    nw = nc * ns
    b_per_w = B // nw
    chunks = b_per_w // _CH
    mesh = plsc.VectorSubcoreMesh(core_axis_name="c", subcore_axis_name="s")

    @functools.partial(
        pl.kernel,
        mesh=mesh,
        out_type=jax.ShapeDtypeStruct((_D, B), jnp.float32),
        scratch_types=[
            pltpu.VMEM((chunks, _CH), jnp.int32),
            pltpu.VMEM((_CH, _D), jnp.float32),
            pltpu.VMEM((_CH, _D), jnp.float32),
            pltpu.VMEM((_D, _CH), jnp.float32),
            pltpu.VMEM((_D, _CH), jnp.float32),
            pltpu.SemaphoreType.DMA,
            pltpu.SemaphoreType.DMA,
            pltpu.SemaphoreType.DMA,
            pltpu.SemaphoreType.DMA,
        ],
        compiler_params=pltpu.CompilerParams(
            use_tc_tiling_on_sc=False, needs_layout_passes=False),
    )
    def lookup(table_hbm, idx_hbm, out_hbm, idx_v,
               gbuf0, gbuf1, obuf0, obuf1, g0, g1, w0, w1):
        c = lax.axis_index("c")
        s = lax.axis_index("s")
        wid = s * nc + c
        i32 = jnp.int32
        iota = lax.iota(i32, 16)
        rows8 = [iota + 16 * ll for ll in range(8)]

        pltpu.sync_copy(idx_hbm.at[wid], idx_v)
        gbufs, obufs = (gbuf0, gbuf1), (obuf0, obuf1)
        gsems, wsems = (g0, g1), (w0, w1)

        def b_fire(j, par):
            pltpu.async_copy(table_hbm.at[idx_v.at[j]], gbufs[par], gsems[par])

        def b_wait_g(par):
            pltpu.make_async_copy(
                table_hbm.at[pl.ds(0, _CH)], gbufs[par], gsems[par]).wait()

        def b_transpose(gbuf, obuf):
            for d in range(_D):
                colv = jnp.full((16,), d, dtype=i32)
                for ll in range(8):
                    obuf[d, pl.ds(ll * 16, 16)] = plsc.load_gather(
                        gbuf, [rows8[ll], colv])

        def b_write(j, par):
            pltpu.async_copy(
                obufs[par],
                out_hbm.at[:, pl.ds((wid * chunks + j) * _CH, _CH)],
                wsems[par])

        def b_drain_w(par):
            pltpu.make_async_copy(
                obufs[par], out_hbm.at[:, pl.ds(0, _CH)], wsems[par]).wait()

        b_fire(0, 0)

        def b_body(t, carry):
            b_fire(2 * t + 1, 1)
            b_wait_g(0)

            @pl.when(t > 0)
            def _():
                b_drain_w(0)

            b_transpose(gbuf0, obuf0)
            b_write(2 * t, 0)

            @pl.when(2 * t + 2 < chunks)
            def _():
                b_fire(2 * t + 2, 0)

            b_wait_g(1)

            @pl.when(t > 0)
            def _():
                b_drain_w(1)

            b_transpose(gbuf1, obuf1)
            b_write(2 * t + 1, 1)
            return carry

        lax.fori_loop(0, chunks // 2, b_body, 0)
        b_drain_w(0)
        b_drain_w(1)

    return lookup


def kernel(x, weight):
    B = x.shape[0] * x.shape[1]
    info = plsc.get_sparse_core_info()
    nw = info.num_cores * info.num_subcores
    idx = x.reshape(nw, (B // nw) // _CH, _CH)
    out_t = _make_lookup(B, weight.shape[0])(weight, idx)
    return out_t.T.reshape(x.shape[0], x.shape[1], _D)


# speedup vs baseline: 1.0744x; 1.0744x over previous
import functools

import jax
import jax.numpy as jnp
from jax import lax
from jax.experimental import pallas as pl
from jax.experimental.pallas import tpu as pltpu
from jax.experimental.pallas import tpu_sc as plsc

_D = 32
_CH = 128


@functools.cache
def _make_lookup(B: int, V: int):
    info = plsc.get_sparse_core_info()
    nc, ns = info.num_cores, info.num_subcores
    nw = nc * ns
    b_per_w = B // nw
    chunks = b_per_w // _CH
    mesh = plsc.VectorSubcoreMesh(core_axis_name="c", subcore_axis_name="s")

    @functools.partial(
        pl.kernel,
        mesh=mesh,
        out_type=jax.ShapeDtypeStruct((_D, B), jnp.float32),
        scratch_types=[
            pltpu.VMEM((chunks, _CH), jnp.int32),
            pltpu.VMEM((_CH, _D), jnp.float32),
            pltpu.VMEM((_CH, _D), jnp.float32),
            pltpu.VMEM((_D, _CH), jnp.float32),
            pltpu.VMEM((_D, _CH), jnp.float32),
            pltpu.SemaphoreType.DMA,
            pltpu.SemaphoreType.DMA,
            pltpu.SemaphoreType.DMA,
            pltpu.SemaphoreType.DMA,
        ],
        compiler_params=pltpu.CompilerParams(
            use_tc_tiling_on_sc=False, needs_layout_passes=False,
            disable_bounds_checks=True),
    )
    def lookup(table_hbm, idx_hbm, out_hbm, idx_v,
               gbuf0, gbuf1, obuf0, obuf1, g0, g1, w0, w1):
        c = lax.axis_index("c")
        s = lax.axis_index("s")
        wid = s * nc + c
        i32 = jnp.int32
        iota = lax.iota(i32, 16)
        rows2 = [iota, iota + 16]

        pltpu.sync_copy(idx_hbm.at[wid], idx_v)
        gbufs, obufs = (gbuf0, gbuf1), (obuf0, obuf1)
        gsems, wsems = (g0, g1), (w0, w1)

        def b_fire(j, par):
            pltpu.async_copy(table_hbm.at[idx_v.at[j]], gbufs[par], gsems[par])

        def b_wait_g(par):
            pltpu.make_async_copy(
                table_hbm.at[pl.ds(0, _CH)], gbufs[par], gsems[par]).wait()

        def b_transpose(gbuf, obuf):
            for l in range(_CH):
                colv = jnp.full((16,), l, dtype=i32)
                for dd in range(2):
                    plsc.store_scatter(
                        obuf, [rows2[dd], colv], gbuf[l, pl.ds(dd * 16, 16)])

        def b_write(j, par):
            pltpu.async_copy(
                obufs[par],
                out_hbm.at[:, pl.ds((wid * chunks + j) * _CH, _CH)],
                wsems[par])

        def b_drain_w(par):
            pltpu.make_async_copy(
                obufs[par], out_hbm.at[:, pl.ds(0, _CH)], wsems[par]).wait()

        b_fire(0, 0)

        def b_body(t, carry):
            b_fire(2 * t + 1, 1)
            b_wait_g(0)

            @pl.when(t > 0)
            def _():
                b_drain_w(0)

            b_transpose(gbuf0, obuf0)
            b_write(2 * t, 0)

            @pl.when(2 * t + 2 < chunks)
            def _():
                b_fire(2 * t + 2, 0)

            b_wait_g(1)

            @pl.when(t > 0)
            def _():
                b_drain_w(1)

            b_transpose(gbuf1, obuf1)
            b_write(2 * t + 1, 1)
            return carry

        lax.fori_loop(0, chunks // 2, b_body, 0)
        b_drain_w(0)
        b_drain_w(1)

    return lookup


def kernel(x, weight):
    B = x.shape[0] * x.shape[1]
    info = plsc.get_sparse_core_info()
    nw = info.num_cores * info.num_subcores
    idx = x.reshape(nw, (B // nw) // _CH, _CH)
    out_t = _make_lookup(B, weight.shape[0])(weight, idx)
    return out_t.T.reshape(x.shape[0], x.shape[1], _D)


# speedup vs baseline: 1.0967x; 1.0207x over previous
import functools

import jax
import jax.numpy as jnp
from jax import lax
from jax.experimental import pallas as pl
from jax.experimental.pallas import tpu as pltpu
from jax.experimental.pallas import tpu_sc as plsc

_D = 32
_CH = 128


@functools.cache
def _make_lookup(B: int, V: int):
    info = plsc.get_sparse_core_info()
    nc, ns = info.num_cores, info.num_subcores
    nw = nc * ns
    b_per_w = B // nw
    chunks = b_per_w // _CH
    mesh = plsc.VectorSubcoreMesh(core_axis_name="c", subcore_axis_name="s")

    @functools.partial(
        pl.kernel,
        mesh=mesh,
        out_type=jax.ShapeDtypeStruct((_D, B), jnp.float32),
        scratch_types=[
            pltpu.VMEM((chunks, _CH), jnp.int32),
            pltpu.VMEM((_CH, _D), jnp.float32),
            pltpu.VMEM((_CH, _D), jnp.float32),
            pltpu.VMEM((_D, _CH), jnp.float32),
            pltpu.VMEM((_D, _CH), jnp.float32),
            pltpu.SemaphoreType.DMA,
            pltpu.SemaphoreType.DMA,
            pltpu.SemaphoreType.DMA,
            pltpu.SemaphoreType.DMA,
        ],
        compiler_params=pltpu.CompilerParams(
            use_tc_tiling_on_sc=False, needs_layout_passes=False,
            disable_bounds_checks=True),
    )
    def lookup(table_hbm, idx_hbm, out_hbm, idx_v,
               gbuf0, gbuf1, obuf0, obuf1, g0, g1, w0, w1):
        c = lax.axis_index("c")
        s = lax.axis_index("s")
        wid = s * nc + c
        i32 = jnp.int32
        iota = lax.iota(i32, 16)
        rows2 = [iota, iota + 16]

        pltpu.sync_copy(idx_hbm.at[wid], idx_v)
        gbufs, obufs = (gbuf0, gbuf1), (obuf0, obuf1)
        gsems, wsems = (g0, g1), (w0, w1)

        def b_fire(j, par):
            pltpu.async_copy(table_hbm.at[idx_v.at[j]], gbufs[par], gsems[par])

        def b_wait_g(par):
            pltpu.make_async_copy(
                table_hbm.at[pl.ds(0, _CH)], gbufs[par], gsems[par]).wait()

        cols_l = [jnp.full((16,), l, dtype=i32) for l in range(_CH)]

        def b_transpose(gbuf, obuf):
            for lg in range(16):
                vecs = []
                for li in range(8):
                    l = lg * 8 + li
                    for dd in range(2):
                        vecs.append((l, dd, gbuf[l, pl.ds(dd * 16, 16)]))
                for l, dd, v in vecs:
                    plsc.store_scatter(obuf, [rows2[dd], cols_l[l]], v)

        def b_write(j, par):
            pltpu.async_copy(
                obufs[par],
                out_hbm.at[:, pl.ds((wid * chunks + j) * _CH, _CH)],
                wsems[par])

        def b_drain_w(par):
            pltpu.make_async_copy(
                obufs[par], out_hbm.at[:, pl.ds(0, _CH)], wsems[par]).wait()

        b_fire(0, 0)

        def b_body(t, carry):
            b_fire(2 * t + 1, 1)
            b_wait_g(0)

            @pl.when(t > 0)
            def _():
                b_drain_w(0)

            b_transpose(gbuf0, obuf0)
            b_write(2 * t, 0)

            @pl.when(2 * t + 2 < chunks)
            def _():
                b_fire(2 * t + 2, 0)

            b_wait_g(1)

            @pl.when(t > 0)
            def _():
                b_drain_w(1)

            b_transpose(gbuf1, obuf1)
            b_write(2 * t + 1, 1)
            return carry

        lax.fori_loop(0, chunks // 2, b_body, 0)
        b_drain_w(0)
        b_drain_w(1)

    return lookup


def kernel(x, weight):
    B = x.shape[0] * x.shape[1]
    info = plsc.get_sparse_core_info()
    nw = info.num_cores * info.num_subcores
    idx = x.reshape(nw, (B // nw) // _CH, _CH)
    out_t = _make_lookup(B, weight.shape[0])(weight, idx)
    return out_t.T.reshape(x.shape[0], x.shape[1], _D)
